# baseline (speedup 1.0000x reference)
import math

import jax
import jax.numpy as jnp
from jax.experimental import pallas as pl
from jax.experimental.pallas import tpu as pltpu

_BC = 131072
_NSLOT = 3


def kernel(mem, vals, idx):
    limit, feat = mem.shape
    batch = vals.shape[0]
    mem_t = mem.T
    vals_t = vals.T
    bc = min(_BC, math.gcd(limit, batch))
    nb = limit // bc

    idx32 = jnp.asarray(idx, dtype=jnp.int32)
    idx_arr = jnp.atleast_1d(idx32)

    def main_body(sp_ref, mem_ref, vals_ref, out_ref, vbuf, sems):
        i = pl.program_id(0)
        start = pl.multiple_of(sp_ref[0], 128)

        def ov(j):
            return (j * bc < start + batch) & ((j + 1) * bc > start)

        def src0(j):
            return pl.multiple_of(jnp.clip(j * bc - start, 0, batch - bc), 128)

        def vdma(j, slot):
            return pltpu.make_async_copy(
                vals_ref.at[:, pl.ds(src0(j), bc)], vbuf.at[slot], sems.at[slot]
            )

        @pl.when(ov(i) & (i == 0))
        def _():
            vdma(i, i % _NSLOT).start()

        nxt = i + 1

        @pl.when((nxt < nb) & ov(nxt))
        def _():
            vdma(nxt, nxt % _NSLOT).start()

        @pl.when(ov(i))
        def _():
            vdma(i, i % _NSLOT).wait()
            shift = jnp.remainder(src0(i) - (i * bc - start), bc)
            win = pltpu.roll(vbuf[i % _NSLOT], shift, axis=1)
            col = i * bc + jax.lax.broadcasted_iota(jnp.int32, (feat, bc), 1)
            maskv = (col >= start) & (col < start + batch)
            out_ref[...] = jnp.where(maskv, win, mem_ref[...])

        @pl.when(jnp.logical_not(ov(i)))
        def _():
            out_ref[...] = mem_ref[...]

    def frozen_map(i, sp_ref):
        start = sp_ref[0]
        ws = start // bc
        inside = (i * bc >= start) & ((i + 1) * bc <= start + batch)
        return (0, jnp.where(inside, ws, i))

    new_mem_t = pl.pallas_call(
        main_body,
        grid_spec=pltpu.PrefetchScalarGridSpec(
            num_scalar_prefetch=1,
            grid=(nb,),
            in_specs=[
                pl.BlockSpec((feat, bc), frozen_map),
                pl.BlockSpec(memory_space=pltpu.MemorySpace.HBM),
            ],
            out_specs=pl.BlockSpec((feat, bc), lambda i, sp_ref: (0, i)),
            scratch_shapes=[
                pltpu.VMEM((_NSLOT, feat, bc), mem.dtype),
                pltpu.SemaphoreType.DMA((_NSLOT,)),
            ],
        ),
        out_shape=jax.ShapeDtypeStruct((feat, limit), mem.dtype),
    )(idx_arr, mem_t, vals_t)

    new_idx = idx32 + batch
    return (new_mem_t.T, new_idx)


# speedup vs baseline: 1.0505x; 1.0505x over previous
import math

import jax
import jax.numpy as jnp
from jax.experimental import pallas as pl
from jax.experimental.pallas import tpu as pltpu

_BC = 131072
_NSLOT = 3


def kernel(mem, vals, idx):
    limit, feat = mem.shape
    batch = vals.shape[0]
    mem_t = mem.T
    vals_t = vals.T
    bc = min(_BC, math.gcd(limit, batch))
    nb = limit // bc

    idx32 = jnp.asarray(idx, dtype=jnp.int32)
    idx_arr = jnp.atleast_1d(idx32)

    def main_body(sp_ref, mem_ref, vals_ref, out_ref, vbuf, sems):
        i = pl.program_id(0)
        start = pl.multiple_of(sp_ref[0], 128)

        def ov(j):
            return (j * bc < start + batch) & ((j + 1) * bc > start)

        def src0(j):
            return pl.multiple_of(jnp.clip(j * bc - start, 0, batch - bc), 128)

        def vdma(j, slot):
            return pltpu.make_async_copy(
                vals_ref.at[:, pl.ds(src0(j), bc)], vbuf.at[slot], sems.at[slot]
            )

        @pl.when(ov(i) & (i == 0))
        def _():
            vdma(i, i % _NSLOT).start()

        nxt = i + 1

        @pl.when((nxt < nb) & ov(nxt))
        def _():
            vdma(nxt, nxt % _NSLOT).start()

        interior = (i * bc >= start) & ((i + 1) * bc <= start + batch)

        @pl.when(interior)
        def _():
            vdma(i, i % _NSLOT).wait()
            out_ref[...] = vbuf[i % _NSLOT]

        @pl.when(ov(i) & jnp.logical_not(interior))
        def _():
            vdma(i, i % _NSLOT).wait()
            shift = jnp.remainder(src0(i) - (i * bc - start), bc)
            win = pltpu.roll(vbuf[i % _NSLOT], shift, axis=1)
            col = i * bc + jax.lax.broadcasted_iota(jnp.int32, (feat, bc), 1)
            maskv = (col >= start) & (col < start + batch)
            out_ref[...] = jnp.where(maskv, win, mem_ref[...])

        @pl.when(jnp.logical_not(ov(i)))
        def _():
            out_ref[...] = mem_ref[...]

    def frozen_map(i, sp_ref):
        start = sp_ref[0]
        ws = start // bc
        inside = (i * bc >= start) & ((i + 1) * bc <= start + batch)
        return (0, jnp.where(inside, ws, i))

    new_mem_t = pl.pallas_call(
        main_body,
        grid_spec=pltpu.PrefetchScalarGridSpec(
            num_scalar_prefetch=1,
            grid=(nb,),
            in_specs=[
                pl.BlockSpec((feat, bc), frozen_map),
                pl.BlockSpec(memory_space=pltpu.MemorySpace.HBM),
            ],
            out_specs=pl.BlockSpec((feat, bc), lambda i, sp_ref: (0, i)),
            scratch_shapes=[
                pltpu.VMEM((_NSLOT, feat, bc), mem.dtype),
                pltpu.SemaphoreType.DMA((_NSLOT,)),
            ],
        ),
        out_shape=jax.ShapeDtypeStruct((feat, limit), mem.dtype),
    )(idx_arr, mem_t, vals_t)

    new_idx = idx32 + batch
    return (new_mem_t.T, new_idx)


# speedup vs baseline: 1.0815x; 1.0295x over previous
import math

import jax
import jax.numpy as jnp
from jax.experimental import pallas as pl
from jax.experimental.pallas import tpu as pltpu

_BC = 131072
_NSLOT = 3


def kernel(mem, vals, idx):
    limit, feat = mem.shape
    batch = vals.shape[0]
    mem_t = mem.T
    vals_t = vals.T
    bc = min(_BC, math.gcd(limit, batch))
    nb = limit // bc

    idx32 = jnp.asarray(idx, dtype=jnp.int32)
    idx_arr = jnp.atleast_1d(idx32)

    def main_body(sp_ref, mem_ref, vals_ref, out_ref, vbuf, sems):
        i = pl.program_id(0)
        start = pl.multiple_of(sp_ref[0], 128)

        def interior(j):
            return (j * bc >= start) & ((j + 1) * bc <= start + batch)

        def vdma(j, slot):
            src0 = pl.multiple_of(j * bc - start, 128)
            return pltpu.make_async_copy(
                vals_ref.at[:, pl.ds(src0, bc)], vbuf.at[slot], sems.at[slot]
            )

        @pl.when(interior(i) & (i == 0))
        def _():
            vdma(i, i % _NSLOT).start()

        nxt = i + 1

        @pl.when((nxt < nb) & interior(nxt))
        def _():
            vdma(nxt, nxt % _NSLOT).start()

        @pl.when(interior(i))
        def _():
            vdma(i, i % _NSLOT).wait()
            out_ref[...] = vbuf[i % _NSLOT]

        @pl.when(jnp.logical_not(interior(i)))
        def _():
            out_ref[...] = mem_ref[...]

    def frozen_map(i, sp_ref):
        start = sp_ref[0]
        ws = start // bc
        inside = (i * bc >= start) & ((i + 1) * bc <= start + batch)
        return (0, jnp.where(inside, ws, i))

    filled = pl.pallas_call(
        main_body,
        grid_spec=pltpu.PrefetchScalarGridSpec(
            num_scalar_prefetch=1,
            grid=(nb,),
            in_specs=[
                pl.BlockSpec((feat, bc), frozen_map),
                pl.BlockSpec(memory_space=pltpu.MemorySpace.HBM),
            ],
            out_specs=pl.BlockSpec((feat, bc), lambda i, sp_ref: (0, i)),
            scratch_shapes=[
                pltpu.VMEM((_NSLOT, feat, bc), mem.dtype),
                pltpu.SemaphoreType.DMA((_NSLOT,)),
            ],
        ),
        out_shape=jax.ShapeDtypeStruct((feat, limit), mem.dtype),
    )(idx_arr, mem_t, vals_t)

    def fix_body(idx_ref, src_ref, vals_ref, out_ref, vbuf, lsem, ssem):
        start = pl.multiple_of(idx_ref[0], 128)
        srcs = (0, batch - bc)
        dsts = (start, start + batch - bc)
        loads = [
            pltpu.make_async_copy(
                vals_ref.at[:, pl.ds(srcs[k], bc)], vbuf.at[k], lsem.at[k]
            )
            for k in range(2)
        ]
        stores = [
            pltpu.make_async_copy(
                vbuf.at[k], out_ref.at[:, pl.ds(dsts[k], bc)], ssem.at[k]
            )
            for k in range(2)
        ]
        loads[0].start()
        loads[1].start()
        loads[0].wait()
        stores[0].start()
        loads[1].wait()
        stores[1].start()
        stores[0].wait()
        stores[1].wait()

    new_mem_t = pl.pallas_call(
        fix_body,
        in_specs=[
            pl.BlockSpec(memory_space=pltpu.MemorySpace.SMEM),
            pl.BlockSpec(memory_space=pltpu.MemorySpace.HBM),
            pl.BlockSpec(memory_space=pltpu.MemorySpace.HBM),
        ],
        out_specs=pl.BlockSpec(memory_space=pltpu.MemorySpace.HBM),
        out_shape=jax.ShapeDtypeStruct((feat, limit), mem.dtype),
        input_output_aliases={1: 0},
        scratch_shapes=[
            pltpu.VMEM((2, feat, bc), mem.dtype),
            pltpu.SemaphoreType.DMA((2,)),
            pltpu.SemaphoreType.DMA((2,)),
        ],
    )(idx_arr, filled, vals_t)

    new_idx = idx32 + batch
    return (new_mem_t.T, new_idx)
